# baseline (speedup 1.0000x reference)
import functools

import jax
import jax.numpy as jnp
from jax.experimental import pallas as pl
from jax.experimental.pallas import tpu as pltpu
from jax.experimental.pallas import tpu_sc as plsc

_SHAPE = (4, 4, 10)


@functools.partial(
    pl.kernel,
    out_type=jax.ShapeDtypeStruct(_SHAPE, jnp.float32),
    mesh=plsc.VectorSubcoreMesh(
        core_axis_name="c", subcore_axis_name="s", num_cores=1, num_subcores=1
    ),
    compiler_params=pltpu.CompilerParams(needs_layout_passes=False),
    scratch_types=[
        pltpu.VMEM(_SHAPE, jnp.float32),
        pltpu.VMEM((16,), jnp.float32),
        pltpu.VMEM((16,), jnp.int32),
        pltpu.VMEM((16,), jnp.int32),
        pltpu.SemaphoreType.DMA,
        pltpu.SemaphoreType.DMA,
        pltpu.SemaphoreType.DMA,
        pltpu.SemaphoreType.DMA,
    ],
)
def _vec_update(upd_hbm, i1_hbm, i2_hbm, p_hbm, out_hbm,
                buf, upd_v, i1_v, i2_v, sem0, sem1, sem2, sem3):
    cp = pltpu.async_copy(p_hbm, buf, sem0)
    cu = pltpu.async_copy(upd_hbm, upd_v.at[pl.ds(0, 2)], sem1)
    c1 = pltpu.async_copy(i1_hbm, i1_v.at[pl.ds(0, 2)], sem2)
    c2 = pltpu.async_copy(i2_hbm, i2_v.at[pl.ds(0, 2)], sem3)
    cu.wait()
    c1.wait()
    c2.wait()
    cp.wait()
    lane = jnp.arange(16, dtype=jnp.int32)
    plsc.store_scatter(
        buf, [i1_v[...], lane + 1, i2_v[...]], upd_v[...], mask=lane < 2
    )
    pltpu.sync_copy(buf, out_hbm)


def kernel(update, index1, index2, params):
    return _vec_update(
        update, index1.astype(jnp.int32), index2.astype(jnp.int32), params
    )


# speedup vs baseline: 1.0566x; 1.0566x over previous
import functools

import jax
import jax.numpy as jnp
from jax.experimental import pallas as pl
from jax.experimental.pallas import tpu as pltpu
from jax.experimental.pallas import tpu_sc as plsc

_SHAPE = (4, 4, 10)


@functools.partial(
    pl.kernel,
    out_type=jax.ShapeDtypeStruct(_SHAPE, jnp.float32),
    mesh=plsc.ScalarSubcoreMesh(axis_name="c", num_cores=1),
    compiler_params=pltpu.CompilerParams(
        needs_layout_passes=False, skip_device_barrier=True
    ),
    scratch_types=[
        pltpu.SMEM(_SHAPE, jnp.float32),
        pltpu.SMEM((2,), jnp.float32),
        pltpu.SMEM((2,), jnp.int32),
        pltpu.SMEM((2,), jnp.int32),
        pltpu.SemaphoreType.DMA,
        pltpu.SemaphoreType.DMA,
        pltpu.SemaphoreType.DMA,
        pltpu.SemaphoreType.DMA,
    ],
)
def _scs_update(upd_hbm, i1_hbm, i2_hbm, p_hbm, out_hbm,
                p_s, upd_s, i1_s, i2_s, sem0, sem1, sem2, sem3):
    cp = pltpu.async_copy(p_hbm, p_s, sem0)
    cu = pltpu.async_copy(upd_hbm, upd_s, sem1)
    c1 = pltpu.async_copy(i1_hbm, i1_s, sem2)
    c2 = pltpu.async_copy(i2_hbm, i2_s, sem3)
    cu.wait()
    c1.wait()
    c2.wait()
    cp.wait()
    for j in range(2):
        p_s[i1_s[j], j + 1, i2_s[j]] = upd_s[j]
    pltpu.sync_copy(p_s, out_hbm)


def kernel(update, index1, index2, params):
    return _scs_update(
        update, index1.astype(jnp.int32), index2.astype(jnp.int32), params
    )


# speedup vs baseline: 1.0590x; 1.0023x over previous
import functools

import jax
import jax.numpy as jnp
from jax.experimental import pallas as pl
from jax.experimental.pallas import tpu as pltpu
from jax.experimental.pallas import tpu_sc as plsc

_SHAPE = (4, 4, 10)


@functools.partial(
    pl.kernel,
    out_type=jax.ShapeDtypeStruct(_SHAPE, jnp.float32),
    mesh=plsc.ScalarSubcoreMesh(axis_name="c", num_cores=1),
    scratch_types=[
        pltpu.SMEM(_SHAPE, jnp.float32),
        pltpu.SMEM((2,), jnp.float32),
        pltpu.SMEM((2,), jnp.int32),
        pltpu.SMEM((2,), jnp.int32),
        pltpu.SemaphoreType.DMA,
        pltpu.SemaphoreType.DMA,
        pltpu.SemaphoreType.DMA,
        pltpu.SemaphoreType.DMA,
    ],
)
def _scs_update(upd_hbm, i1_hbm, i2_hbm, p_hbm, out_hbm,
                p_s, upd_s, i1_s, i2_s, sem0, sem1, sem2, sem3):
    cp = pltpu.async_copy(p_hbm, p_s, sem0)
    cu = pltpu.async_copy(upd_hbm, upd_s, sem1)
    c1 = pltpu.async_copy(i1_hbm, i1_s, sem2)
    c2 = pltpu.async_copy(i2_hbm, i2_s, sem3)
    cu.wait()
    c1.wait()
    c2.wait()
    cp.wait()
    for j in range(2):
        p_s[i1_s[j], j + 1, i2_s[j]] = upd_s[j]
    pltpu.sync_copy(p_s, out_hbm)


def kernel(update, index1, index2, params):
    return _scs_update(
        update, index1.astype(jnp.int32), index2.astype(jnp.int32), params
    )
